# baseline (speedup 1.0000x reference)
import jax
import jax.numpy as jnp
from jax.experimental import pallas as pl
from jax.experimental.pallas import tpu as pltpu

_LN_EPS = 1e-5
_LEAKY_SLOPE = 0.01


def _gat_tile_kernel(x_ref, wqk_ref, bqk_ref, wv_ref, bv_ref, o_ref):
    bt, J, D = x_ref.shape
    M = bt * J

    x = x_ref[...].reshape(M, D)

    mu = jnp.mean(x, axis=-1, keepdims=True)
    msq = jnp.mean(x * x, axis=-1, keepdims=True)
    rstd = jax.lax.rsqrt(msq - mu * mu + _LN_EPS)
    y = (x - mu) * rstd

    qk = jnp.dot(y, wqk_ref[...], preferred_element_type=jnp.float32)
    qk = (qk + bqk_ref[...]).reshape(bt, J, 2 * D)
    q = qk[..., :D]
    k = qk[..., D:]

    v = jnp.dot(y.astype(jnp.bfloat16), wv_ref[...],
                preferred_element_type=jnp.float32)
    v = (v + bv_ref[...]).reshape(bt, J, D)

    s = jnp.einsum("bqd,bkd->bqk", q, k,
                   preferred_element_type=jnp.float32)
    p = jnp.exp(s)
    r = 1.0 / jnp.sum(p, axis=-1, keepdims=True)
    att = jnp.einsum("bqk,bkd->bqd", p.astype(jnp.bfloat16),
                     v.astype(jnp.bfloat16),
                     preferred_element_type=jnp.float32)
    att = att * r

    act = jnp.maximum(att, _LEAKY_SLOPE * att)
    o_ref[...] = (act.reshape(M, D) + x).reshape(bt, J, D).astype(o_ref.dtype)


def kernel(x, gamma, beta, wqkv, bqkv):
    B, J, D = x.shape
    BT = 64
    grid_b = B // BT

    g = gamma.reshape(D, 1)
    wg = g * wqkv
    bf = beta.reshape(1, D) @ wqkv + bqkv
    wqk = wg[:, :2 * D]
    bqk = bf[:, :2 * D]
    wv = wg[:, 2 * D:].astype(jnp.bfloat16)
    bv = bf[:, 2 * D:]

    fixed = lambda b: (0, 0)

    return pl.pallas_call(
        _gat_tile_kernel,
        out_shape=jax.ShapeDtypeStruct((B, J, D), x.dtype),
        grid=(grid_b,),
        in_specs=[
            pl.BlockSpec((BT, J, D), lambda b: (b, 0, 0)),
            pl.BlockSpec((D, 2 * D), fixed),
            pl.BlockSpec((1, 2 * D), fixed),
            pl.BlockSpec((D, D), fixed),
            pl.BlockSpec((1, D), fixed),
        ],
        out_specs=pl.BlockSpec((BT, J, D), lambda b: (b, 0, 0)),
        compiler_params=pltpu.CompilerParams(
            dimension_semantics=("parallel",)),
    )(x, wqk, bqk, wv, bv)


# speedup vs baseline: 1.0568x; 1.0568x over previous
import jax
import jax.numpy as jnp
from jax.experimental import pallas as pl
from jax.experimental.pallas import tpu as pltpu

_LN_EPS = 1e-5
_LEAKY_SLOPE = 0.01


def _gat_tile_kernel(x_ref, wqkv_ref, bqkv_ref, o_ref):
    bt, J, D = x_ref.shape
    M = bt * J

    x = x_ref[...].reshape(M, D)

    mu = jnp.mean(x, axis=-1, keepdims=True)
    msq = jnp.mean(x * x, axis=-1, keepdims=True)
    rstd = jax.lax.rsqrt(msq - mu * mu + _LN_EPS)
    y = (x - mu) * rstd

    qkv = jnp.dot(y, wqkv_ref[...], preferred_element_type=jnp.float32)
    qkv = (qkv + bqkv_ref[...]).reshape(bt, J, 3 * D)
    q = qkv[..., :D]
    k = qkv[..., D:2 * D]
    v = qkv[..., 2 * D:]

    s = jnp.einsum("bqd,bkd->bqk", q, k,
                   preferred_element_type=jnp.float32)
    p = jnp.exp(s)
    r = 1.0 / jnp.sum(p, axis=-1, keepdims=True)
    att = jnp.einsum("bqk,bkd->bqd", p, v,
                     preferred_element_type=jnp.float32)
    att = att * r

    act = jnp.maximum(att, _LEAKY_SLOPE * att)
    o_ref[...] = (act.reshape(M, D) + x).reshape(bt, J, D).astype(o_ref.dtype)


def kernel(x, gamma, beta, wqkv, bqkv):
    B, J, D = x.shape
    BT = 64
    grid_b = B // BT

    wg = gamma.reshape(D, 1) * wqkv
    bf = beta.reshape(1, D) @ wqkv + bqkv

    fixed = lambda b: (0, 0)

    return pl.pallas_call(
        _gat_tile_kernel,
        out_shape=jax.ShapeDtypeStruct((B, J, D), x.dtype),
        grid=(grid_b,),
        in_specs=[
            pl.BlockSpec((BT, J, D), lambda b: (b, 0, 0)),
            pl.BlockSpec((D, 3 * D), fixed),
            pl.BlockSpec((1, 3 * D), fixed),
        ],
        out_specs=pl.BlockSpec((BT, J, D), lambda b: (b, 0, 0)),
        compiler_params=pltpu.CompilerParams(
            dimension_semantics=("parallel",)),
    )(x, wg, bf)


# speedup vs baseline: 1.2338x; 1.1675x over previous
import jax
import jax.numpy as jnp
from jax.experimental import pallas as pl
from jax.experimental.pallas import tpu as pltpu

_LN_EPS = 1e-5
_LEAKY_SLOPE = 0.01


def _gat_tile_kernel(x_ref, gamma_ref, beta_ref, wqkv_ref, bqkv_ref, o_ref):
    bt, J, D = x_ref.shape
    M = bt * J

    x = x_ref[...].reshape(M, D)

    mu = jnp.mean(x, axis=-1, keepdims=True)
    msq = jnp.mean(x * x, axis=-1, keepdims=True)
    rstd = jax.lax.rsqrt(msq - mu * mu + _LN_EPS)
    xn = (x - mu) * rstd
    xn = xn * gamma_ref[...] + beta_ref[...]

    qkv = jnp.dot(xn, wqkv_ref[...], preferred_element_type=jnp.float32)
    qkv = qkv + bqkv_ref[...]
    qkv = qkv.reshape(bt, J, 3 * D)
    q = qkv[..., :D]
    k = qkv[..., D:2 * D]
    v = qkv[..., 2 * D:]

    s = jnp.einsum("bqd,bkd->bqk", q, k,
                   preferred_element_type=jnp.float32)
    p = jnp.exp(s)
    r = 1.0 / jnp.sum(p, axis=-1, keepdims=True)
    att = jnp.einsum("bqk,bkd->bqd", p, v,
                     preferred_element_type=jnp.float32)
    att = att * r

    act = jnp.maximum(att, _LEAKY_SLOPE * att)
    o_ref[...] = (act.reshape(M, D) + x).reshape(bt, J, D).astype(o_ref.dtype)


def kernel(x, gamma, beta, wqkv, bqkv):
    B, J, D = x.shape
    BT = 64
    grid_b = B // BT

    fixed = lambda b: (0, 0)

    return pl.pallas_call(
        _gat_tile_kernel,
        out_shape=jax.ShapeDtypeStruct((B, J, D), x.dtype),
        grid=(grid_b,),
        in_specs=[
            pl.BlockSpec((BT, J, D), lambda b: (b, 0, 0)),
            pl.BlockSpec((1, D), fixed),
            pl.BlockSpec((1, D), fixed),
            pl.BlockSpec((D, 3 * D), fixed),
            pl.BlockSpec((1, 3 * D), fixed),
        ],
        out_specs=pl.BlockSpec((BT, J, D), lambda b: (b, 0, 0)),
        compiler_params=pltpu.CompilerParams(
            dimension_semantics=("parallel",)),
    )(x, gamma, beta, wqkv, bqkv)


# speedup vs baseline: 1.2744x; 1.0329x over previous
import jax
import jax.numpy as jnp
from jax.experimental import pallas as pl
from jax.experimental.pallas import tpu as pltpu

_LN_EPS = 1e-5
_LEAKY_SLOPE = 0.01


def _gat_tile_kernel(x_ref, gamma_ref, beta_ref, wqkv_ref, bqkv_ref, o_ref):
    bt, J, D = x_ref.shape
    M = bt * J

    x = x_ref[...].reshape(M, D)

    mu = jnp.mean(x, axis=-1, keepdims=True)
    msq = jnp.mean(x * x, axis=-1, keepdims=True)
    rstd = jax.lax.rsqrt(msq - mu * mu + _LN_EPS)
    xn = (x - mu) * rstd
    xn = xn * gamma_ref[...] + beta_ref[...]

    qkv = jnp.dot(xn, wqkv_ref[...], preferred_element_type=jnp.float32)
    qkv = qkv + bqkv_ref[...]
    qkv = qkv.reshape(bt, J, 3 * D)
    q = qkv[..., :D]
    k = qkv[..., D:2 * D]
    v = qkv[..., 2 * D:]

    s = jnp.einsum("bqd,bkd->bqk", q, k,
                   preferred_element_type=jnp.float32)
    p = jnp.exp(s)
    r = 1.0 / jnp.sum(p, axis=-1, keepdims=True)
    att = jnp.einsum("bqk,bkd->bqd", p, v,
                     preferred_element_type=jnp.float32)
    att = att * r

    act = jnp.maximum(att, _LEAKY_SLOPE * att)
    o_ref[...] = (act.reshape(M, D) + x).reshape(bt, J, D).astype(o_ref.dtype)


def kernel(x, gamma, beta, wqkv, bqkv):
    B, J, D = x.shape
    BT = 128
    grid_b = B // BT

    fixed = lambda b: (0, 0)

    return pl.pallas_call(
        _gat_tile_kernel,
        out_shape=jax.ShapeDtypeStruct((B, J, D), x.dtype),
        grid=(grid_b,),
        in_specs=[
            pl.BlockSpec((BT, J, D), lambda b: (b, 0, 0)),
            pl.BlockSpec((1, D), fixed),
            pl.BlockSpec((1, D), fixed),
            pl.BlockSpec((D, 3 * D), fixed),
            pl.BlockSpec((1, 3 * D), fixed),
        ],
        out_specs=pl.BlockSpec((BT, J, D), lambda b: (b, 0, 0)),
        compiler_params=pltpu.CompilerParams(
            dimension_semantics=("parallel",)),
    )(x, gamma, beta, wqkv, bqkv)
